# baseline (speedup 1.0000x reference)
import jax
import jax.numpy as jnp
from jax.experimental import pallas as pl
from jax.experimental.pallas import tpu as pltpu

_TAU_INV = 2.0
_CH = 4


def _chunk_body(l_ref, u_ref, o_ref):
    for c in range(_CH):
        a = jnp.exp(l_ref[pl.ds(c, 1), :] * _TAU_INV)
        t = jnp.log(u_ref[c])
        aw = a / (t * t)
        s = jnp.sum(aw, axis=1, keepdims=True)
        o_ref[pl.ds(c, 1), :] = jnp.max(
            aw * (1.0 / s), axis=0, keepdims=True)


def kernel(logits, uniform):
    B, K, D = uniform.shape
    outs = []
    for i in range(B // _CH):
        u_i = jax.lax.slice_in_dim(uniform, i * _CH, (i + 1) * _CH, axis=0)
        l_i = jax.lax.slice_in_dim(logits, i * _CH, (i + 1) * _CH, axis=0)
        outs.append(pl.pallas_call(
            _chunk_body,
            in_specs=[
                pl.BlockSpec(memory_space=pltpu.VMEM),
                pl.BlockSpec(memory_space=pltpu.VMEM),
            ],
            out_specs=pl.BlockSpec(memory_space=pltpu.VMEM),
            out_shape=jax.ShapeDtypeStruct((_CH, D), jnp.float32),
        )(l_i, u_i))
    return jnp.concatenate(outs, axis=0)


# speedup vs baseline: 2.2043x; 2.2043x over previous
import jax
import jax.numpy as jnp
from jax.experimental import pallas as pl
from jax.experimental.pallas import tpu as pltpu

_TAU_INV = 2.0
_SLAB = 4
_NBUF = 2
_NSPLIT = 2
_NCHUNK = 4


def _u_copy(u_hbm, buf, sems, g, slot, j):
    n = _SLAB // _NSPLIT
    return pltpu.make_async_copy(
        u_hbm.at[pl.ds(g * _SLAB + j * n, n)],
        buf.at[slot, pl.ds(j * n, n)],
        sems.at[slot, j],
    )


def _body(l_hbm, u_hbm, o_hbm, buf, lbuf, obuf, sems, lsems, osems):
    B, K, D = u_hbm.shape
    nslab = B // _SLAB

    for r in range(_NBUF):
        for j in range(_NSPLIT):
            _u_copy(u_hbm, buf, sems, r, r, j).start(priority=j % 2)
        pltpu.make_async_copy(
            l_hbm.at[pl.ds(r * _SLAB, _SLAB)], lbuf.at[r],
            lsems.at[r]).start()

    def step(g, carry):
        slot = jax.lax.rem(g, _NBUF)
        for j in range(_NSPLIT):
            _u_copy(u_hbm, buf, sems, g, slot, j).wait()
        pltpu.make_async_copy(
            l_hbm.at[pl.ds(g * _SLAB, _SLAB)], lbuf.at[slot],
            lsems.at[slot]).wait()

        @pl.when(g >= _NBUF)
        def _drain_prev():
            pltpu.make_async_copy(
                obuf.at[slot], o_hbm.at[pl.ds((g - _NBUF) * _SLAB, _SLAB)],
                osems.at[slot]).wait()

        Dc = D // _NCHUNK
        for c in range(_SLAB):
            a = jnp.exp(lbuf[slot, c] * _TAU_INV)
            ars = []
            s = None
            for i in range(_NCHUNK):
                t = jnp.log(buf[slot, c, :, i * Dc:(i + 1) * Dc])
                ar = a[:, i * Dc:(i + 1) * Dc] / (t * t)
                ars.append(ar)
                p = jnp.sum(ar, axis=1, keepdims=True)
                s = p if s is None else s + p
            r_ = 1.0 / s
            obuf[slot, c] = jnp.concatenate(
                [jnp.max(ar * r_, axis=0, keepdims=True) for ar in ars],
                axis=1)

        pltpu.make_async_copy(
            obuf.at[slot], o_hbm.at[pl.ds(g * _SLAB, _SLAB)],
            osems.at[slot]).start()

        g2 = g + _NBUF

        @pl.when(g2 < nslab)
        def _refill():
            slot2 = jax.lax.rem(g2, _NBUF)
            for j in range(_NSPLIT):
                _u_copy(u_hbm, buf, sems, g2, slot2, j).start(priority=j % 2)
            pltpu.make_async_copy(
                l_hbm.at[pl.ds(g2 * _SLAB, _SLAB)], lbuf.at[slot2],
                lsems.at[slot2]).start()

        return carry

    jax.lax.fori_loop(0, nslab, step, 0)

    for t in range(_NBUF):
        g = nslab - _NBUF + t
        pltpu.make_async_copy(
            obuf.at[g % _NBUF], o_hbm.at[pl.ds(g * _SLAB, _SLAB)],
            osems.at[g % _NBUF]).wait()


def kernel(logits, uniform):
    B, K, D = uniform.shape
    out = pl.pallas_call(
        _body,
        in_specs=[
            pl.BlockSpec(memory_space=pltpu.HBM),
            pl.BlockSpec(memory_space=pltpu.HBM),
        ],
        out_specs=pl.BlockSpec(memory_space=pltpu.HBM),
        out_shape=jax.ShapeDtypeStruct((B, 1, D), jnp.float32),
        scratch_shapes=[
            pltpu.VMEM((_NBUF, _SLAB, K, D), jnp.float32),
            pltpu.VMEM((_NBUF, _SLAB, 1, D), jnp.float32),
            pltpu.VMEM((_NBUF, _SLAB, 1, D), jnp.float32),
            pltpu.SemaphoreType.DMA((_NBUF, _NSPLIT)),
            pltpu.SemaphoreType.DMA((_NBUF,)),
            pltpu.SemaphoreType.DMA((_NBUF,)),
        ],
    )(logits.reshape(B, 1, D), uniform)
    return out.reshape(B, D)


# speedup vs baseline: 2.2222x; 1.0081x over previous
import jax
import jax.numpy as jnp
from jax.experimental import pallas as pl
from jax.experimental.pallas import tpu as pltpu

_TAU_INV = 2.0
_NBUF = 4
_NSPLIT = 2
_NOUT = 4
_NCHUNK = 4


def _u_copy(u_hbm, buf, sems, row, slot, j, D):
    Ds = D // _NSPLIT
    return pltpu.make_async_copy(
        u_hbm.at[row, :, pl.ds(j * Ds, Ds)],
        buf.at[slot, :, pl.ds(j * Ds, Ds)],
        sems.at[slot, j],
    )


def _body(l_hbm, u_hbm, o_hbm, buf, lbuf, obuf, sems, lsems, osems):
    B, K, D = u_hbm.shape

    for r in range(_NBUF):
        for j in range(_NSPLIT):
            _u_copy(u_hbm, buf, sems, r, r, j, D).start(priority=j % 2)
        pltpu.make_async_copy(l_hbm.at[r], lbuf.at[r], lsems.at[r]).start()

    def step(b, carry):
        slot = jax.lax.rem(b, _NBUF)
        for j in range(_NSPLIT):
            _u_copy(u_hbm, buf, sems, b, slot, j, D).wait()
        pltpu.make_async_copy(l_hbm.at[b], lbuf.at[slot], lsems.at[slot]).wait()

        a = jnp.exp(lbuf[slot] * _TAU_INV)
        Dc = D // _NCHUNK
        ars = []
        s = None
        for i in range(_NCHUNK):
            t = jnp.log(buf[slot, :, i * Dc:(i + 1) * Dc])
            ar = a[:, i * Dc:(i + 1) * Dc] / (t * t)
            ars.append(ar)
            p = jnp.sum(ar, axis=1, keepdims=True)
            s = p if s is None else s + p
        r_ = 1.0 / s
        m = jnp.concatenate(
            [jnp.max(ar * r_, axis=0, keepdims=True) for ar in ars],
            axis=1)

        oslot = jax.lax.rem(b, _NOUT)

        @pl.when(b >= _NOUT)
        def _drain_prev():
            pltpu.make_async_copy(
                obuf.at[oslot], o_hbm.at[b - _NOUT], osems.at[oslot]).wait()

        obuf[oslot] = m
        pltpu.make_async_copy(
            obuf.at[oslot], o_hbm.at[b], osems.at[oslot]).start()

        b2 = b + _NBUF

        @pl.when(b2 < B)
        def _refill():
            slot2 = jax.lax.rem(b2, _NBUF)
            for j in range(_NSPLIT):
                _u_copy(u_hbm, buf, sems, b2, slot2, j, D).start(
                    priority=j % 2)
            pltpu.make_async_copy(
                l_hbm.at[b2], lbuf.at[slot2], lsems.at[slot2]).start()

        return carry

    jax.lax.fori_loop(0, B, step, 0)

    for t in range(_NOUT):
        row = B - _NOUT + t
        pltpu.make_async_copy(
            obuf.at[row % _NOUT], o_hbm.at[row],
            osems.at[row % _NOUT]).wait()


def kernel(logits, uniform):
    B, K, D = uniform.shape
    out = pl.pallas_call(
        _body,
        in_specs=[
            pl.BlockSpec(memory_space=pltpu.HBM),
            pl.BlockSpec(memory_space=pltpu.HBM),
        ],
        out_specs=pl.BlockSpec(memory_space=pltpu.HBM),
        out_shape=jax.ShapeDtypeStruct((B, 1, D), jnp.float32),
        scratch_shapes=[
            pltpu.VMEM((_NBUF, K, D), jnp.float32),
            pltpu.VMEM((_NBUF, 1, D), jnp.float32),
            pltpu.VMEM((_NOUT, 1, D), jnp.float32),
            pltpu.SemaphoreType.DMA((_NBUF, _NSPLIT)),
            pltpu.SemaphoreType.DMA((_NBUF,)),
            pltpu.SemaphoreType.DMA((_NOUT,)),
        ],
    )(logits.reshape(B, 1, D), uniform)
    return out.reshape(B, D)
